# baseline (speedup 1.0000x reference)
import jax
import jax.numpy as jnp
from jax import lax
from jax.experimental import pallas as pl
from jax.experimental.pallas import tpu as pltpu

_N = 4194304
_LANES = 256
_RLANES = 128
_ROWS = (2 * _N) // _LANES
_BLOCK_R = 256
_GRID = _ROWS // _BLOCK_R

_C2 = float(jnp.pi) * 2.0
_C4 = float(jnp.pi) * 4.0
_HALF_PI = float(jnp.pi) / 2.0


def _body(ws_ref, rdn_ref, wo_ref, g_ref, out_ref, d_ref):
    @pl.when(pl.program_id(0) == 0)
    def _():
        j = lax.broadcasted_iota(jnp.int32, (_RLANES, _LANES), 0)
        l = lax.broadcasted_iota(jnp.int32, (_RLANES, _LANES), 1)
        d_ref[...] = (j == lax.div(l, 2)).astype(jnp.bfloat16)

    w0 = ws_ref[0, 0]
    w1 = ws_ref[0, 1]
    wm = jnp.maximum(w0, w1)
    shp = (_BLOCK_R, _RLANES)
    e0 = jnp.exp(jnp.full(shp, w0 - wm, jnp.float32))
    e1 = jnp.exp(jnp.full(shp, w1 - wm, jnp.float32))
    p = e1 / (e0 + e1)

    msk = (rdn_ref[...] < p).astype(jnp.bfloat16)
    mexp = lax.dot_general(msk, d_ref[...], (((1,), (0,)), ((), ())),
                           preferred_element_type=jnp.float32)
    lam = mexp > 0.5

    v = wo_ref[...]
    even = (lax.broadcasted_iota(jnp.int32, v.shape, 1) & 1) == 0
    pn = jnp.where(even, pltpu.roll(v, _LANES - 1, 1), pltpu.roll(v, 1, 1))

    x2 = v * 2.0 - 1.0
    p2 = pn * 2.0 - 1.0
    xv = jnp.where(even, x2, p2)
    yv = jnp.where(even, p2, x2)
    av = jnp.abs(xv)
    ay = jnp.abs(yv)
    xdom = av > ay
    nz = jnp.maximum(av, ay) > 0.0
    cond2 = jnp.logical_and(jnp.logical_not(xdom), nz)

    num = jnp.where(xdom, yv, xv)
    den = jnp.where(xdom, xv, jnp.where(cond2, yv, 1.0))
    ratio = (jnp.float32(_C4) * num) / den
    theta = jnp.where(xdom, ratio, jnp.float32(_C2) - ratio)
    r = jnp.where(xdom, xv, yv)
    phase = jnp.where(even, 0.0, jnp.float32(_HALF_PI))
    z_lam = r * jnp.cos(theta - phase)

    z_gauss = g_ref[...] * jnp.sqrt(jnp.float32(0.1))
    out_ref[...] = jnp.where(lam, z_lam, z_gauss)


def kernel(weight_scores, rdn, wo, gauss_base):
    rdn2 = rdn.reshape(_ROWS, _RLANES)
    wo2 = wo.reshape(_ROWS, _LANES)
    g2 = gauss_base.reshape(_ROWS, _LANES)
    out = pl.pallas_call(
        _body,
        grid=(_GRID,),
        in_specs=[
            pl.BlockSpec(memory_space=pltpu.SMEM),
            pl.BlockSpec((_BLOCK_R, _RLANES), lambda i: (i, 0)),
            pl.BlockSpec((_BLOCK_R, _LANES), lambda i: (i, 0)),
            pl.BlockSpec((_BLOCK_R, _LANES), lambda i: (i, 0)),
        ],
        out_specs=pl.BlockSpec((_BLOCK_R, _LANES), lambda i: (i, 0)),
        out_shape=jax.ShapeDtypeStruct((_ROWS, _LANES), jnp.float32),
        scratch_shapes=[pltpu.VMEM((_RLANES, _LANES), jnp.bfloat16)],
    )(weight_scores, rdn2, wo2, g2)
    return out.reshape(_N, 2)


# speedup vs baseline: 1.0007x; 1.0007x over previous
import jax
import jax.numpy as jnp
from jax import lax
from jax.experimental import pallas as pl
from jax.experimental.pallas import tpu as pltpu

_N = 4194304
_L = 128
_RROWS = _N // _L
_IROWS = (2 * _N) // _L
_BR = 256
_GRID = _RROWS // _BR

_C2 = float(jnp.pi) * 2.0
_C4 = float(jnp.pi) * 4.0
_HALF_PI = float(jnp.pi) / 2.0


def _body(ws_ref, rdn_ref, wo_ref, g_ref, out_ref, d_ref):
    @pl.when(pl.program_id(0) == 0)
    def _():
        j = lax.broadcasted_iota(jnp.int32, (_L, 2 * _L), 0)
        l = lax.broadcasted_iota(jnp.int32, (_L, 2 * _L), 1)
        d_ref[...] = (j == lax.div(l, 2)).astype(jnp.bfloat16)

    w0 = ws_ref[0, 0]
    w1 = ws_ref[0, 1]
    wm = jnp.maximum(w0, w1)
    shp = (_BR, _L)
    e0 = jnp.exp(jnp.full(shp, w0 - wm, jnp.float32))
    e1 = jnp.exp(jnp.full(shp, w1 - wm, jnp.float32))
    p = e1 / (e0 + e1)

    msk = (rdn_ref[...] < p).astype(jnp.bfloat16)
    mexp = lax.dot_general(msk, d_ref[...], (((1,), (0,)), ((), ())),
                           preferred_element_type=jnp.float32)
    lam = mexp.reshape(2 * _BR, _L) > 0.5

    v = wo_ref[...]
    even = (lax.broadcasted_iota(jnp.int32, v.shape, 1) & 1) == 0
    pn = jnp.where(even, pltpu.roll(v, _L - 1, 1), pltpu.roll(v, 1, 1))

    x2 = v * 2.0 - 1.0
    p2 = pn * 2.0 - 1.0
    xv = jnp.where(even, x2, p2)
    yv = jnp.where(even, p2, x2)
    av = jnp.abs(xv)
    ay = jnp.abs(yv)
    xdom = av > ay
    nz = jnp.maximum(av, ay) > 0.0
    cond2 = jnp.logical_and(jnp.logical_not(xdom), nz)

    num = jnp.where(xdom, yv, xv)
    den = jnp.where(xdom, xv, jnp.where(cond2, yv, 1.0))
    ratio = (jnp.float32(_C4) * num) / den
    theta = jnp.where(xdom, ratio, jnp.float32(_C2) - ratio)
    r = jnp.where(xdom, xv, yv)
    phase = jnp.where(even, 0.0, jnp.float32(_HALF_PI))
    z_lam = r * jnp.cos(theta - phase)

    z_gauss = g_ref[...] * jnp.sqrt(jnp.float32(0.1))
    out_ref[...] = jnp.where(lam, z_lam, z_gauss)


def kernel(weight_scores, rdn, wo, gauss_base):
    rdn2 = rdn.reshape(_RROWS, _L)
    wo2 = wo.reshape(_IROWS, _L)
    g2 = gauss_base.reshape(_IROWS, _L)
    out = pl.pallas_call(
        _body,
        grid=(_GRID,),
        in_specs=[
            pl.BlockSpec(memory_space=pltpu.SMEM),
            pl.BlockSpec((_BR, _L), lambda i: (i, 0)),
            pl.BlockSpec((2 * _BR, _L), lambda i: (i, 0)),
            pl.BlockSpec((2 * _BR, _L), lambda i: (i, 0)),
        ],
        out_specs=pl.BlockSpec((2 * _BR, _L), lambda i: (i, 0)),
        out_shape=jax.ShapeDtypeStruct((_IROWS, _L), jnp.float32),
        scratch_shapes=[pltpu.VMEM((_L, 2 * _L), jnp.bfloat16)],
    )(weight_scores, rdn2, wo2, g2)
    return out.reshape(_N, 2)


# speedup vs baseline: 104.7080x; 104.6318x over previous
import jax
import jax.numpy as jnp
from jax import lax
from jax.experimental import pallas as pl
from jax.experimental.pallas import tpu as pltpu

_N = 4194304
_L = 128
_RROWS = _N // _L
_IROWS = 2 * _RROWS
_BR = 256
_GRID = _RROWS // _BR

_C2 = float(jnp.pi) * 2.0
_C4 = float(jnp.pi) * 4.0


def _body(ws_ref, rdn_ref, wo_ref, g_ref, out_ref):
    w0 = ws_ref[0, 0]
    w1 = ws_ref[0, 1]
    wm = jnp.maximum(w0, w1)
    shp = (_BR, _L)
    e0 = jnp.exp(jnp.full(shp, w0 - wm, jnp.float32))
    e1 = jnp.exp(jnp.full(shp, w1 - wm, jnp.float32))
    p = e1 / (e0 + e1)
    m = rdn_ref[...] < p

    ex = pl.Slice(0, _BR, 2)
    ey = pl.Slice(1, _BR, 2)
    x = wo_ref[ex, :] * 2.0 - 1.0
    y = wo_ref[ey, :] * 2.0 - 1.0

    ax = jnp.abs(x)
    ay = jnp.abs(y)
    cond1 = ax > ay
    nz = jnp.maximum(ax, ay) > 0.0
    cond2 = jnp.logical_and(jnp.logical_not(cond1), nz)

    num = jnp.where(cond1, y, x)
    den = jnp.where(cond1, x, jnp.where(cond2, y, 1.0))
    ratio = (jnp.float32(_C4) * num) / den
    theta = jnp.where(cond1, ratio, jnp.float32(_C2) - ratio)
    r = jnp.where(cond1, x, y)

    s = jnp.sqrt(jnp.float32(0.1))
    out_ref[ex, :] = jnp.where(m, r * jnp.cos(theta), g_ref[ex, :] * s)
    out_ref[ey, :] = jnp.where(m, r * jnp.sin(theta), g_ref[ey, :] * s)


def _pairs_to_rows(a):
    return a.reshape(_RROWS, _L, 2).transpose(0, 2, 1).reshape(_IROWS, _L)


def kernel(weight_scores, rdn, wo, gauss_base):
    rdn2 = rdn.reshape(_RROWS, _L)
    wo2 = _pairs_to_rows(wo)
    g2 = _pairs_to_rows(gauss_base)
    out = pl.pallas_call(
        _body,
        grid=(_GRID,),
        in_specs=[
            pl.BlockSpec(memory_space=pltpu.SMEM),
            pl.BlockSpec((_BR, _L), lambda i: (i, 0)),
            pl.BlockSpec((2 * _BR, _L), lambda i: (i, 0)),
            pl.BlockSpec((2 * _BR, _L), lambda i: (i, 0)),
        ],
        out_specs=pl.BlockSpec((2 * _BR, _L), lambda i: (i, 0)),
        out_shape=jax.ShapeDtypeStruct((_IROWS, _L), jnp.float32),
    )(weight_scores, rdn2, wo2, g2)
    return out.reshape(_RROWS, 2, _L).transpose(0, 2, 1).reshape(_N, 2)


# speedup vs baseline: 126.0084x; 1.2034x over previous
import jax
import jax.numpy as jnp
from jax import lax
from jax.experimental import pallas as pl
from jax.experimental.pallas import tpu as pltpu

_N = 4194304
_L = 128
_RROWS = _N // _L
_IROWS = 2 * _RROWS
_BR = 256
_GRID = _RROWS // _BR

_COS_C = [1.0, -19.739206314086914, 64.93917083740234, -85.45116424560547,
          60.17622375488281, -26.000497817993164, 6.575565814971924]
_SIN_C = [6.2831854820251465, -41.34170150756836, 81.60515594482422,
          -76.70345306396484, 42.029598236083984, -14.91390609741211,
          3.258183240890503]


def _horner(z, coeffs):
    acc = jnp.float32(coeffs[-1])
    for c in coeffs[-2::-1]:
        acc = acc * z + jnp.float32(c)
    return acc


def _body(ws_ref, rdn_ref, wo_ref, g_ref, out_ref):
    w0 = ws_ref[0, 0]
    w1 = ws_ref[0, 1]
    wm = jnp.maximum(w0, w1)
    shp = (_BR, _L)
    e0 = jnp.exp(jnp.full(shp, w0 - wm, jnp.float32))
    e1 = jnp.exp(jnp.full(shp, w1 - wm, jnp.float32))
    p = e1 / (e0 + e1)
    m = rdn_ref[...] < p

    ex = pl.Slice(0, _BR, 2)
    ey = pl.Slice(1, _BR, 2)
    x = wo_ref[ex, :] * 2.0 - 1.0
    y = wo_ref[ey, :] * 2.0 - 1.0

    ax = jnp.abs(x)
    ay = jnp.abs(y)
    cond1 = ax > ay
    nz = jnp.maximum(ax, ay) > 0.0
    cond2 = jnp.logical_and(jnp.logical_not(cond1), nz)

    num = jnp.where(cond1, y, x)
    den = jnp.where(cond1, x, jnp.where(cond2, y, 1.0))
    u2 = (num / den) * 2.0
    w = u2 - jnp.round(u2)
    z = w * w
    cosv = _horner(z, _COS_C)
    sinv = w * _horner(z, _SIN_C)
    r = jnp.where(cond1, x, y)

    s = jnp.sqrt(jnp.float32(0.1))
    out_ref[ex, :] = jnp.where(m, r * cosv, g_ref[ex, :] * s)
    out_ref[ey, :] = jnp.where(m, jnp.where(cond1, r, -r) * sinv,
                               g_ref[ey, :] * s)


def _pairs_to_rows(a):
    return a.reshape(_RROWS, _L, 2).transpose(0, 2, 1).reshape(_IROWS, _L)


def kernel(weight_scores, rdn, wo, gauss_base):
    rdn2 = rdn.reshape(_RROWS, _L)
    wo2 = _pairs_to_rows(wo)
    g2 = _pairs_to_rows(gauss_base)
    out = pl.pallas_call(
        _body,
        grid=(_GRID,),
        in_specs=[
            pl.BlockSpec(memory_space=pltpu.SMEM),
            pl.BlockSpec((_BR, _L), lambda i: (i, 0)),
            pl.BlockSpec((2 * _BR, _L), lambda i: (i, 0)),
            pl.BlockSpec((2 * _BR, _L), lambda i: (i, 0)),
        ],
        out_specs=pl.BlockSpec((2 * _BR, _L), lambda i: (i, 0)),
        out_shape=jax.ShapeDtypeStruct((_IROWS, _L), jnp.float32),
    )(weight_scores, rdn2, wo2, g2)
    return out.reshape(_RROWS, 2, _L).transpose(0, 2, 1).reshape(_N, 2)
